# baseline (speedup 1.0000x reference)
import jax
import jax.numpy as jnp
from jax import lax
from jax.experimental import pallas as pl
from jax.experimental.pallas import tpu as pltpu
from jax.experimental.pallas import tpu_sc as plsc

B, S, D = 4, 8192, 1024
NC, NS = 2, 16
NW = NC * NS
POS_W = S // NW
R = 8
NBLK = POS_W // R
UNROLL = 4


def _body(x_hbm, w_hbm, out_hbm, *scr):
    wb = [scr[2 * s] for s in range(3)]
    xb = [scr[2 * s + 1] for s in range(3)]
    in_sem = [scr[6 + s] for s in range(3)]
    out_sem = [scr[9 + s] for s in range(3)]

    wid = lax.axis_index("s") * NC + lax.axis_index("c")
    base = wid * POS_W

    def start_in(s, j):
        r0 = base + j * R
        pltpu.async_copy(w_hbm.at[pl.ds(r0, R)], wb[s], in_sem[s])
        pltpu.async_copy(x_hbm.at[:, pl.ds(r0, R)], xb[s], in_sem[s])

    def start_out(s, j):
        r0 = base + j * R
        pltpu.async_copy(xb[s], out_hbm.at[:, pl.ds(r0, R)], out_sem[s])

    def wait_in(s):
        pltpu.make_async_copy(w_hbm.at[pl.ds(0, R)], wb[s], in_sem[s]).wait()
        pltpu.make_async_copy(x_hbm.at[:, pl.ds(0, R)], xb[s], in_sem[s]).wait()

    def wait_out(s):
        pltpu.make_async_copy(xb[s], out_hbm.at[:, pl.ds(0, R)], out_sem[s]).wait()

    def compute(s):
        x0 = xb[s]
        w = wb[s]

        def add_chunk(i, c2):
            o = i * (16 * UNROLL)
            for u in range(UNROLL):
                sl = pl.ds(o + u * 16, 16)
                for r in range(R):
                    wv = w[r, sl]
                    for b in range(B):
                        x0[b, r, sl] = x0[b, r, sl] + wv
            return c2

        lax.fori_loop(0, D // (16 * UNROLL), add_chunk, 0)

    def process(j, s, refill, wait_prev):
        wait_in(s)
        compute(s)
        start_out(s, j)
        if refill:
            s2 = (s + 2) % 3
            if wait_prev:
                wait_out(s2)
            start_in(s2, j + 2)

    start_in(0, 0)
    start_in(1, 1)
    process(0, 0, True, False)
    process(1, 1, True, True)

    def middle(t, c):
        j = 2 + 3 * t
        process(j, 2, True, True)
        process(j + 1, 0, True, True)
        process(j + 2, 1, True, True)
        return c

    lax.fori_loop(0, (NBLK - 5) // 3, middle, 0)
    process(NBLK - 3, 2, True, True)
    process(NBLK - 2, 0, False, False)
    process(NBLK - 1, 1, False, False)
    wait_out(2)
    wait_out(0)
    wait_out(1)


@jax.jit
def _pe_add(x, w2):
    mesh = plsc.VectorSubcoreMesh(core_axis_name="c", subcore_axis_name="s")
    bufs = []
    for _ in range(3):
        bufs.append(pltpu.VMEM((R, D), jnp.float32))
        bufs.append(pltpu.VMEM((B, R, D), jnp.float32))
    f = pl.kernel(
        _body,
        out_type=jax.ShapeDtypeStruct((B, S, D), jnp.float32),
        mesh=mesh,
        scratch_types=bufs + [pltpu.SemaphoreType.DMA] * 6,
        compiler_params=pltpu.CompilerParams(use_tc_tiling_on_sc=True),
    )
    return f(x, w2)


def kernel(x, weight):
    return _pe_add(x, weight.reshape(S, D))


# speedup vs baseline: 2.6554x; 2.6554x over previous
import jax
import jax.numpy as jnp
from jax import lax
from jax.experimental import pallas as pl
from jax.experimental.pallas import tpu as pltpu
from jax.experimental.pallas import tpu_sc as plsc

B, S, D = 4, 8192, 1024
NC, NS = 2, 16
NW = NC * NS
POS_W = S // NW
R = 8
BLK = R * D
NBLK = POS_W // R
UNROLL = 8


def _body(x_hbm, w_hbm, out_hbm, *scr):
    wb = [scr[5 * s] for s in range(3)]
    xb = [scr[5 * s + 1:5 * s + 5] for s in range(3)]
    in_sem = [scr[15 + s] for s in range(3)]
    out_sem = [scr[18 + s] for s in range(3)]

    wid = lax.axis_index("s") * NC + lax.axis_index("c")
    base = wid * POS_W

    def start_in(s, j):
        r0 = base + j * R
        pltpu.async_copy(w_hbm.at[pl.ds(r0, R)], wb[s], in_sem[s])
        for b in range(B):
            pltpu.async_copy(
                x_hbm.at[pl.ds(b * S + r0, R)], xb[s][b], in_sem[s])

    def start_out(s, j):
        r0 = base + j * R
        for b in range(B):
            pltpu.async_copy(
                xb[s][b], out_hbm.at[pl.ds(b * S + r0, R)], out_sem[s])

    def wait_in(s):
        pltpu.make_async_copy(w_hbm.at[pl.ds(0, R)], wb[s], in_sem[s]).wait()
        for b in range(B):
            pltpu.make_async_copy(
                x_hbm.at[pl.ds(0, R)], xb[s][b], in_sem[s]).wait()

    def wait_out(s):
        for b in range(B):
            pltpu.make_async_copy(
                xb[s][b], out_hbm.at[pl.ds(0, R)], out_sem[s]).wait()

    def compute(s):
        x0, x1, x2, x3 = xb[s]
        w = wb[s]

        def add_chunk(i, c2):
            o = i * (16 * UNROLL)
            for u in range(UNROLL):
                sl = pl.ds(o + u * 16, 16)
                for r in range(R):
                    wv = w[r, sl]
                    x0[r, sl] = x0[r, sl] + wv
                    x1[r, sl] = x1[r, sl] + wv
                    x2[r, sl] = x2[r, sl] + wv
                    x3[r, sl] = x3[r, sl] + wv
            return c2

        lax.fori_loop(0, D // (16 * UNROLL), add_chunk, 0)

    def process(j, s, refill, wait_prev):
        wait_in(s)
        compute(s)
        start_out(s, j)
        if refill:
            s2 = (s + 2) % 3
            if wait_prev:
                wait_out(s2)
            start_in(s2, j + 2)

    start_in(0, 0)
    start_in(1, 1)
    process(0, 0, True, False)
    process(1, 1, True, True)

    def middle(t, c):
        j = 2 + 3 * t
        process(j, 2, True, True)
        process(j + 1, 0, True, True)
        process(j + 2, 1, True, True)
        return c

    lax.fori_loop(0, (NBLK - 5) // 3, middle, 0)
    process(NBLK - 3, 2, True, True)
    process(NBLK - 2, 0, False, False)
    process(NBLK - 1, 1, False, False)
    wait_out(2)
    wait_out(0)
    wait_out(1)


@jax.jit
def _pe_add(x2, w2):
    mesh = plsc.VectorSubcoreMesh(core_axis_name="c", subcore_axis_name="s")
    bufs = [pltpu.VMEM((R, D), jnp.float32) for _ in range(3 * (1 + B))]
    f = pl.kernel(
        _body,
        out_type=jax.ShapeDtypeStruct((B * S, D), jnp.float32),
        mesh=mesh,
        scratch_types=bufs + [pltpu.SemaphoreType.DMA] * 6,
        compiler_params=pltpu.CompilerParams(use_tc_tiling_on_sc=True),
    )
    return f(x2, w2)


def kernel(x, weight):
    out = _pe_add(x.reshape(B * S, D), weight.reshape(S, D))
    return out.reshape(x.shape)
